# baseline (speedup 1.0000x reference)
import jax
import jax.numpy as jnp
from jax import lax
from jax.experimental import pallas as pl
from jax.experimental.pallas import tpu as pltpu
from jax.experimental.pallas import tpu_sc as plsc

BATCH = 4
SEQ = 2048
HIDDEN = 768
LANES = 16
NCHUNK = HIDDEN // LANES

NC = 2
NS = 16
NW = NC * NS
POS_PER_W = SEQ // NW


def _rsqrt_newton(a):
    i = lax.bitcast_convert_type(a, jnp.int32)
    i = jnp.int32(0x5F3759DF) - lax.shift_right_logical(i, 1)
    y = lax.bitcast_convert_type(i, jnp.float32)
    half_a = 0.5 * a
    for _ in range(3):
        y = y * (1.5 - half_a * y * y)
    return y


def _butterfly_perms():
    lane = lax.iota(jnp.int32, LANES)
    return [lane ^ k for k in (8, 4, 2, 1)]


_GATHER_DNUMS = lax.GatherDimensionNumbers(
    offset_dims=(), collapsed_slice_dims=(0,), start_index_map=(0,))


def _lane_gather(x, idx):
    return lax.gather(x, idx[:, None], _GATHER_DNUMS, (1,),
                      mode=lax.GatherScatterMode.PROMISE_IN_BOUNDS)


def _lane_allsum(x, perms):
    for p in perms:
        x = x + _lane_gather(x, p)
    return x


def _sc_embed_ln(ids_hbm, tok_hbm, pos_hbm, gamma_hbm, beta_hbm, out_hbm,
                 idx_v, tok_v, pos_v, gam_v, bet_v, sem):
    wid = lax.axis_index("s") * NC + lax.axis_index("c")
    p0 = wid * POS_PER_W

    pltpu.sync_copy(gamma_hbm, gam_v)
    pltpu.sync_copy(beta_hbm, bet_v)
    pltpu.sync_copy(pos_hbm.at[pl.ds(p0, POS_PER_W)], pos_v)

    inv_h = jnp.float32(1.0 / HIDDEN)
    eps = jnp.float32(1e-12)
    perms = _butterfly_perms()

    for b in range(BATCH):
        base = b * SEQ + p0
        pltpu.sync_copy(ids_hbm.at[pl.ds(base, POS_PER_W)], idx_v)
        pltpu.async_copy(tok_hbm.at[idx_v], tok_v, sem).wait()

        def row_body(r, _):
            acc = jnp.zeros((LANES,), jnp.float32)
            acc2 = jnp.zeros((LANES,), jnp.float32)
            for j in range(NCHUNK):
                sl = pl.ds(j * LANES, LANES)
                x = tok_v[r, sl] + pos_v[r, sl]
                acc = acc + x
                acc2 = acc2 + x * x
            mu = _lane_allsum(acc, perms) * inv_h
            var = _lane_allsum(acc2, perms) * inv_h - mu * mu
            rstd = _rsqrt_newton(var + eps)
            for j in range(NCHUNK):
                sl = pl.ds(j * LANES, LANES)
                x = tok_v[r, sl] + pos_v[r, sl]
                tok_v[r, sl] = (x - mu) * rstd * gam_v[sl] + bet_v[sl]
            return _

        lax.fori_loop(0, POS_PER_W, row_body, None)
        pltpu.sync_copy(tok_v, out_hbm.at[pl.ds(base, POS_PER_W)])


@jax.jit
def kernel(input_ids, token_table, pos_table, gamma, beta):
    ids_flat = input_ids.reshape(-1)
    mesh = plsc.VectorSubcoreMesh(core_axis_name="c", subcore_axis_name="s")
    out = pl.kernel(
        _sc_embed_ln,
        mesh=mesh,
        out_type=jax.ShapeDtypeStruct((BATCH * SEQ, HIDDEN), jnp.float32),
        scratch_types=[
            pltpu.VMEM((POS_PER_W,), jnp.int32),
            pltpu.VMEM((POS_PER_W, HIDDEN), jnp.float32),
            pltpu.VMEM((POS_PER_W, HIDDEN), jnp.float32),
            pltpu.VMEM((HIDDEN,), jnp.float32),
            pltpu.VMEM((HIDDEN,), jnp.float32),
            pltpu.SemaphoreType.DMA,
        ],
    )(ids_flat, token_table, pos_table, gamma, beta)
    return out.reshape(BATCH, SEQ, HIDDEN)
